# Initial kernel scaffold; baseline (speedup 1.0000x reference)
import jax
import jax.numpy as jnp
from jax.experimental import pallas as pl


def kernel(input, mask, embed, W1, b1, W2, b2, Wf, bf):
    raise NotImplementedError("write your pallas kernel here")


# speedup vs baseline: 1.5695x; 1.5695x over previous
import jax
import jax.numpy as jnp
from jax.experimental import pallas as pl
from jax.experimental.pallas import tpu as pltpu

B, S, RAW = 16384, 50, 109
EMB = 20
H = 512
NB = 10
OUT = 20
PACK = 4
PW = PACK * EMB

BP = 128
BM = 1024


def _pool_kernel(in_ref, mask_ref, rot_ref, q_ref, emb_ref, out_ref,
                 raw_tile, q_smem, sem):
    cp = pltpu.make_async_copy(q_ref, q_smem, sem)
    cp.start()
    cp.wait()

    def row_body(r, carry):
        for s in range(S):
            raw_tile[r, s] = emb_ref[q_smem[r, s], 0]
        return carry

    jax.lax.fori_loop(0, BP, row_body, 0)

    raw = raw_tile[...]
    idx = rot_ref[...] + jax.lax.broadcasted_iota(
        jnp.int32, (1, 1, EMB), 2)
    e = jnp.take_along_axis(raw, idx, axis=2)

    xin = in_ref[...]
    f = xin[:, :, 1:]
    maskf = mask_ref[...]
    v3 = maskf > 0

    inv = 1.0 / jnp.sum(maskf, axis=1)

    e_sum = jnp.sum(jnp.where(v3, e, 0.0), axis=1)
    f_sum = jnp.sum(jnp.where(v3, f, 0.0), axis=1)
    e_mx = jnp.max(jnp.where(v3, e, -jnp.inf), axis=1)
    f_mx = jnp.max(jnp.where(v3, f, -jnp.inf), axis=1)
    e_mn = jnp.min(jnp.where(v3, e, jnp.inf), axis=1)
    f_mn = jnp.min(jnp.where(v3, f, jnp.inf), axis=1)

    def clip(t):
        return jnp.clip(t, 1e-9, 1e9)

    out_ref[...] = jnp.concatenate([
        e_sum * inv, f_sum * inv,
        clip(e_mx), clip(f_mx),
        clip(e_mn), clip(f_mn),
        e[:, S - 1, :], f[:, S - 1, :],
    ], axis=1)


def _mlp_kernel(h0_ref, w1_ref, b1_ref, w2_ref, b2_ref, wf_ref, bf_ref,
                out_ref, h_s):
    i = pl.program_id(1)

    @pl.when(i == 0)
    def _():
        h_s[...] = h0_ref[...]

    h = h_s[...]
    t = jnp.maximum(
        jnp.dot(h, w1_ref[0], preferred_element_type=jnp.float32)
        + b1_ref[0], 0.0)
    t = jnp.maximum(
        jnp.dot(t, w2_ref[0], preferred_element_type=jnp.float32)
        + b2_ref[0], 0.0)
    h = h + t
    h_s[...] = h

    @pl.when(i == NB - 1)
    def _():
        out_ref[...] = (
            jnp.dot(h, wf_ref[...], preferred_element_type=jnp.float32)
            + bf_ref[...])


def kernel(input, mask, embed, W1, b1, W2, b2, Wf, bf):
    ids = input[:, :, 0].astype(jnp.int32)
    q = ids >> 2
    rot3 = ((ids & 3) * EMB).reshape(B, S, 1)
    maskf3 = mask.astype(jnp.float32).reshape(B, S, 1)
    vocab = embed.shape[0]
    emb3 = embed.reshape(vocab // PACK, 1, PW)

    h0 = pl.pallas_call(
        _pool_kernel,
        grid=(B // BP,),
        in_specs=[
            pl.BlockSpec((BP, S, RAW), lambda j: (j, 0, 0)),
            pl.BlockSpec((BP, S, 1), lambda j: (j, 0, 0)),
            pl.BlockSpec((BP, S, 1), lambda j: (j, 0, 0)),
            pl.BlockSpec((BP, S), lambda j: (j, 0)),
            pl.BlockSpec((vocab // PACK, 1, PW), lambda j: (0, 0, 0)),
        ],
        out_specs=pl.BlockSpec((BP, H), lambda j: (j, 0)),
        out_shape=jax.ShapeDtypeStruct((B, H), jnp.float32),
        scratch_shapes=[
            pltpu.VMEM((BP, S, PW), jnp.float32),
            pltpu.SMEM((BP, S), jnp.int32),
            pltpu.SemaphoreType.DMA,
        ],
        compiler_params=pltpu.CompilerParams(
            dimension_semantics=("parallel",),
            vmem_limit_bytes=56 * 2**20,
        ),
        name="pool",
    )(input, maskf3, rot3, q, emb3)

    out = pl.pallas_call(
        _mlp_kernel,
        grid=(B // BM, NB),
        in_specs=[
            pl.BlockSpec((BM, H), lambda j, i: (j, 0)),
            pl.BlockSpec((1, H, 4 * H), lambda j, i: (i, 0, 0)),
            pl.BlockSpec((1, 1, 4 * H), lambda j, i: (i, 0, 0)),
            pl.BlockSpec((1, 4 * H, H), lambda j, i: (i, 0, 0)),
            pl.BlockSpec((1, 1, H), lambda j, i: (i, 0, 0)),
            pl.BlockSpec((H, OUT), lambda j, i: (0, 0)),
            pl.BlockSpec((1, OUT), lambda j, i: (0, 0)),
        ],
        out_specs=pl.BlockSpec((BM, OUT), lambda j, i: (j, 0)),
        out_shape=jax.ShapeDtypeStruct((B, OUT), jnp.float32),
        scratch_shapes=[pltpu.VMEM((BM, H), jnp.float32)],
        compiler_params=pltpu.CompilerParams(
            dimension_semantics=("parallel", "arbitrary"),
            vmem_limit_bytes=56 * 2**20,
        ),
        name="resmlp",
    )(h0, W1, b1.reshape(NB, 1, 4 * H), W2, b2.reshape(NB, 1, H),
      Wf, bf.reshape(1, OUT))
    return out


# speedup vs baseline: 1.7448x; 1.1117x over previous
import jax
import jax.numpy as jnp
from jax.experimental import pallas as pl
from jax.experimental.pallas import tpu as pltpu

B, S, RAW = 16384, 50, 109
EMB = 20
H = 512
NB = 10
OUT = 20
PACK = 4
PW = PACK * EMB

BP = 128
BM = 1024


def _pool_kernel(in_ref, mask_ref, rot_ref, q_ref, emb_ref, out_ref,
                 raw_tile, q_smem, sem):
    cp = pltpu.make_async_copy(q_ref, q_smem, sem)
    cp.start()
    cp.wait()

    def row_body(r, carry):
        for s in range(S):
            raw_tile[r, s] = emb_ref[q_smem[r, s], 0]
        return carry

    jax.lax.fori_loop(0, BP, row_body, 0)

    raw = raw_tile[...]
    idx = rot_ref[...] + jax.lax.broadcasted_iota(
        jnp.int32, (1, 1, EMB), 2)
    e = jnp.take_along_axis(raw, idx, axis=2)

    xin = in_ref[...]
    x = jnp.concatenate([e, xin[:, :, 1:]], axis=2)

    seq2 = jnp.sum(mask_ref[...], axis=1, keepdims=True)
    inv = 1.0 / seq2
    m3 = jax.lax.broadcasted_iota(
        jnp.int32, (1, S, 1), 1) < seq2.astype(jnp.int32).reshape(BP, 1, 1)
    big = jnp.where(m3, 0.0, -1e30)
    msk = jnp.where(m3, 1.0, 0.0)
    x_sum = jnp.sum(x * msk, axis=1)
    x_mx = jnp.max(x + big, axis=1)
    x_mn = jnp.min(x - big, axis=1)

    def clip(t):
        return jnp.clip(t, 1e-9, 1e9)

    out_ref[...] = jnp.concatenate([
        x_sum * inv, clip(x_mx), clip(x_mn), x[:, S - 1, :],
    ], axis=1)


def _mlp_kernel(h0_ref, w1_ref, b1_ref, w2_ref, b2_ref, wf_ref, bf_ref,
                out_ref, h_s):
    i = pl.program_id(1)

    @pl.when(i == 0)
    def _():
        h_s[...] = h0_ref[...]

    h = h_s[...]
    t = jnp.maximum(
        jnp.dot(h, w1_ref[0], preferred_element_type=jnp.float32)
        + b1_ref[0], 0.0)
    t = jnp.maximum(
        jnp.dot(t, w2_ref[0], preferred_element_type=jnp.float32)
        + b2_ref[0], 0.0)
    h = h + t
    h_s[...] = h

    @pl.when(i == NB - 1)
    def _():
        out_ref[...] = (
            jnp.dot(h, wf_ref[...], preferred_element_type=jnp.float32)
            + bf_ref[...])


def kernel(input, mask, embed, W1, b1, W2, b2, Wf, bf):
    ids = input[:, :, 0].astype(jnp.int32)
    q = ids >> 2
    rot3 = ((ids & 3) * EMB).reshape(B, S, 1)
    maskf = mask.astype(jnp.float32)
    vocab = embed.shape[0]
    emb3 = embed.reshape(vocab // PACK, 1, PW)

    h0 = pl.pallas_call(
        _pool_kernel,
        grid=(B // BP,),
        in_specs=[
            pl.BlockSpec((BP, S, RAW), lambda j: (j, 0, 0)),
            pl.BlockSpec((BP, S), lambda j: (j, 0)),
            pl.BlockSpec((BP, S, 1), lambda j: (j, 0, 0)),
            pl.BlockSpec((BP, S), lambda j: (j, 0)),
            pl.BlockSpec((vocab // PACK, 1, PW), lambda j: (0, 0, 0)),
        ],
        out_specs=pl.BlockSpec((BP, H), lambda j: (j, 0)),
        out_shape=jax.ShapeDtypeStruct((B, H), jnp.float32),
        scratch_shapes=[
            pltpu.VMEM((BP, S, PW), jnp.float32),
            pltpu.SMEM((BP, S), jnp.int32),
            pltpu.SemaphoreType.DMA,
        ],
        compiler_params=pltpu.CompilerParams(
            dimension_semantics=("parallel",),
            vmem_limit_bytes=56 * 2**20,
        ),
        name="pool",
    )(input, maskf, rot3, q, emb3)

    out = pl.pallas_call(
        _mlp_kernel,
        grid=(B // BM, NB),
        in_specs=[
            pl.BlockSpec((BM, H), lambda j, i: (j, 0)),
            pl.BlockSpec((1, H, 4 * H), lambda j, i: (i, 0, 0)),
            pl.BlockSpec((1, 1, 4 * H), lambda j, i: (i, 0, 0)),
            pl.BlockSpec((1, 4 * H, H), lambda j, i: (i, 0, 0)),
            pl.BlockSpec((1, 1, H), lambda j, i: (i, 0, 0)),
            pl.BlockSpec((H, OUT), lambda j, i: (0, 0)),
            pl.BlockSpec((1, OUT), lambda j, i: (0, 0)),
        ],
        out_specs=pl.BlockSpec((BM, OUT), lambda j, i: (j, 0)),
        out_shape=jax.ShapeDtypeStruct((B, OUT), jnp.float32),
        scratch_shapes=[pltpu.VMEM((BM, H), jnp.float32)],
        compiler_params=pltpu.CompilerParams(
            dimension_semantics=("parallel", "arbitrary"),
            vmem_limit_bytes=56 * 2**20,
        ),
        name="resmlp",
    )(h0, W1, b1.reshape(NB, 1, 4 * H), W2, b2.reshape(NB, 1, H),
      Wf, bf.reshape(1, OUT))
    return out


# speedup vs baseline: 1.7556x; 1.0062x over previous
import jax
import jax.numpy as jnp
from jax.experimental import pallas as pl
from jax.experimental.pallas import tpu as pltpu

B, S, RAW = 16384, 50, 109
EMB = 20
H = 512
NB = 10
OUT = 20
PACK = 4
PW = PACK * EMB

BP = 128
BM = 2048
NCHUNK = 2


def _pool_kernel(in_ref, mask_ref, rot_ref, q_ref, emb_ref, out_ref,
                 raw_tile, q_smem, sem):
    cp = pltpu.make_async_copy(q_ref, q_smem, sem)
    cp.start()
    cp.wait()

    def row_body(r, carry):
        for s in range(S):
            raw_tile[r, s] = emb_ref[q_smem[r, s], 0]
        return carry

    jax.lax.fori_loop(0, BP, row_body, 0)

    raw = raw_tile[...]
    idx = rot_ref[...] + jax.lax.broadcasted_iota(
        jnp.int32, (1, 1, EMB), 2)
    e = jnp.take_along_axis(raw, idx, axis=2)

    xin = in_ref[...]
    x = jnp.concatenate([e, xin[:, :, 1:]], axis=2)

    seq2 = jnp.sum(mask_ref[...], axis=1, keepdims=True)
    inv = 1.0 / seq2
    m3 = jax.lax.broadcasted_iota(
        jnp.int32, (1, S, 1), 1) < seq2.astype(jnp.int32).reshape(BP, 1, 1)
    big = jnp.where(m3, 0.0, -1e30)
    msk = jnp.where(m3, 1.0, 0.0)
    x_sum = jnp.sum(x * msk, axis=1)
    x_mx = jnp.max(x + big, axis=1)
    x_mn = jnp.min(x - big, axis=1)

    def clip(t):
        return jnp.clip(t, 1e-9, 1e9)

    out_ref[...] = jnp.concatenate([
        x_sum * inv, clip(x_mx), clip(x_mn), x[:, S - 1, :],
    ], axis=1)


def _mlp_kernel(h0_ref, w1_ref, b1_ref, w2_ref, b2_ref, wf_ref, bf_ref,
                out_ref, h_s):
    i = pl.program_id(1)

    @pl.when(i == 0)
    def _():
        h_s[...] = h0_ref[...]

    h = h_s[...]
    w1 = w1_ref[0]
    w2 = w2_ref[0]
    acc = None
    for c in range(NCHUNK):
        lo, hi = c * (4 * H // NCHUNK), (c + 1) * (4 * H // NCHUNK)
        tc = jnp.maximum(
            jnp.dot(h, w1[:, lo:hi], preferred_element_type=jnp.float32)
            + b1_ref[0, :, lo:hi], 0.0)
        part = jnp.dot(tc, w2[lo:hi, :], preferred_element_type=jnp.float32)
        acc = part if acc is None else acc + part
    h = h + jnp.maximum(acc + b2_ref[0], 0.0)
    h_s[...] = h

    @pl.when(i == NB - 1)
    def _():
        out_ref[...] = (
            jnp.dot(h, wf_ref[...], preferred_element_type=jnp.float32)
            + bf_ref[...])


def kernel(input, mask, embed, W1, b1, W2, b2, Wf, bf):
    ids = input[:, :, 0].astype(jnp.int32)
    q = ids >> 2
    rot3 = ((ids & 3) * EMB).reshape(B, S, 1)
    maskf = mask.astype(jnp.float32)
    vocab = embed.shape[0]
    emb3 = embed.reshape(vocab // PACK, 1, PW)

    h0 = pl.pallas_call(
        _pool_kernel,
        grid=(B // BP,),
        in_specs=[
            pl.BlockSpec((BP, S, RAW), lambda j: (j, 0, 0)),
            pl.BlockSpec((BP, S), lambda j: (j, 0)),
            pl.BlockSpec((BP, S, 1), lambda j: (j, 0, 0)),
            pl.BlockSpec((BP, S), lambda j: (j, 0)),
            pl.BlockSpec((vocab // PACK, 1, PW), lambda j: (0, 0, 0)),
        ],
        out_specs=pl.BlockSpec((BP, H), lambda j: (j, 0)),
        out_shape=jax.ShapeDtypeStruct((B, H), jnp.float32),
        scratch_shapes=[
            pltpu.VMEM((BP, S, PW), jnp.float32),
            pltpu.SMEM((BP, S), jnp.int32),
            pltpu.SemaphoreType.DMA,
        ],
        compiler_params=pltpu.CompilerParams(
            dimension_semantics=("parallel",),
            vmem_limit_bytes=56 * 2**20,
        ),
        name="pool",
    )(input, maskf, rot3, q, emb3)

    out = pl.pallas_call(
        _mlp_kernel,
        grid=(B // BM, NB),
        in_specs=[
            pl.BlockSpec((BM, H), lambda j, i: (j, 0)),
            pl.BlockSpec((1, H, 4 * H), lambda j, i: (i, 0, 0)),
            pl.BlockSpec((1, 1, 4 * H), lambda j, i: (i, 0, 0)),
            pl.BlockSpec((1, 4 * H, H), lambda j, i: (i, 0, 0)),
            pl.BlockSpec((1, 1, H), lambda j, i: (i, 0, 0)),
            pl.BlockSpec((H, OUT), lambda j, i: (0, 0)),
            pl.BlockSpec((1, OUT), lambda j, i: (0, 0)),
        ],
        out_specs=pl.BlockSpec((BM, OUT), lambda j, i: (j, 0)),
        out_shape=jax.ShapeDtypeStruct((B, OUT), jnp.float32),
        scratch_shapes=[pltpu.VMEM((BM, H), jnp.float32)],
        compiler_params=pltpu.CompilerParams(
            dimension_semantics=("parallel", "arbitrary"),
            vmem_limit_bytes=56 * 2**20,
        ),
        name="resmlp",
    )(h0, W1, b1.reshape(NB, 1, 4 * H), W2, b2.reshape(NB, 1, H),
      Wf, bf.reshape(1, OUT))
    return out
